# baseline (speedup 1.0000x reference)
import functools

import jax
import jax.numpy as jnp
from jax import lax
from jax.experimental import pallas as pl
from jax.experimental.pallas import tpu as pltpu
from jax.experimental.pallas import tpu_sc as plsc

_B = 16384
_D = 26
_EMB = 64
_ROW = _D * _EMB
_NC = 2
_NS = 16
_NW = _NC * _NS
_BPW = _B // _NW
_R = 32
_NCHUNK = _BPW // _R
_DP = 32


def _sc_body(x_hbm, w_hbm, v_hbm, vflat_hbm, idx_hbm, emb_hbm, wide_hbm,
             xbuf, vbuf, vfbuf, wbuf, idxbuf, obuf0, obuf1, widebuf,
             wsm, sbuf, qbuf, sem, sem0, sem1):
    cid = lax.axis_index("c")
    sid = lax.axis_index("s")
    wid = sid * _NC + cid
    base = wid * _BPW

    pltpu.sync_copy(x_hbm.at[pl.ds(base * _D, _BPW * _D)], xbuf)
    pltpu.sync_copy(w_hbm, wbuf)
    pltpu.sync_copy(idx_hbm, idxbuf)
    pltpu.sync_copy(vflat_hbm, vfbuf)
    pltpu.async_copy(v_hbm.at[idxbuf], vbuf, sem).wait()

    lanes = lax.iota(jnp.int32, 16)
    wv0 = wbuf[pl.ds(0, 16)]
    wv1 = wbuf[pl.ds(16, 16)]
    s_lo = jnp.zeros((16,), jnp.float32)
    s_hi = jnp.zeros((16,), jnp.float32)
    q_lo = jnp.zeros((16,), jnp.float32)
    q_hi = jnp.zeros((16,), jnp.float32)
    for e in range(_EMB):
        ve_lo = plsc.load_gather(vfbuf, [lanes * _EMB + e])
        ve_hi = plsc.load_gather(vfbuf, [(lanes + 16) * _EMB + e])
        s_lo = s_lo + ve_lo
        s_hi = s_hi + ve_hi
        q_lo = q_lo + ve_lo * ve_lo
        q_hi = q_hi + ve_hi * ve_hi
    for d in range(_D):
        sbuf[d] = s_lo[d] if d < 16 else s_hi[d - 16]
        qbuf[d] = q_lo[d] if d < 16 else q_hi[d - 16]
        wsm[d] = wv0[d] if d < 16 else wv1[d - 16]
    zero16 = jnp.zeros((16,), jnp.float32)

    def compute_chunk(g, obuf):
        row0 = g * _R
        for grp in range(_R // 16):
            r0s = row0 + grp * 16
            rows = r0s + lanes

            @pl.loop(0, _D, init_carry=(zero16, zero16, zero16))
            def _field(d, carry):
                o1, sv, qv = carry
                xv = plsc.load_gather(xbuf, [rows * _D + d])
                v0 = vbuf[d, pl.ds(0, 16)]
                v1 = vbuf[d, pl.ds(16, 16)]
                v2 = vbuf[d, pl.ds(32, 16)]
                v3 = vbuf[d, pl.ds(48, 16)]
                dbase = d * _EMB
                for k in range(16):
                    xbc = plsc.load_gather(
                        xbuf, [jnp.full((16,), (r0s + k) * _D + d, jnp.int32)])
                    ob = (grp * 16 + k) * _ROW + dbase
                    obuf[pl.ds(ob, 16)] = xbc * v0
                    obuf[pl.ds(ob + 16, 16)] = xbc * v1
                    obuf[pl.ds(ob + 32, 16)] = xbc * v2
                    obuf[pl.ds(ob + 48, 16)] = xbc * v3
                o1 = o1 + xv * wsm[d]
                sv = sv + xv * sbuf[d]
                qv = qv + (xv * xv) * qbuf[d]
                return (o1, sv, qv)

            o1, sv, qv = _field
            o2 = 0.5 * (sv * sv - qv)
            plsc.store_scatter(widebuf, [rows * 2], o1)
            plsc.store_scatter(widebuf, [rows * 2 + 1], o2)

    def emb_slice(g):
        return emb_hbm.at[pl.ds((base + g * _R) * _ROW, _R * _ROW)]

    compute_chunk(0, obuf0)
    pltpu.async_copy(obuf0, emb_slice(0), sem0)
    compute_chunk(1, obuf1)
    pltpu.async_copy(obuf1, emb_slice(1), sem1)

    @pl.loop(1, _NCHUNK // 2)
    def _pair(p):
        g = p * 2
        pltpu.make_async_copy(obuf0, emb_slice(0), sem0).wait()
        compute_chunk(g, obuf0)
        pltpu.async_copy(obuf0, emb_slice(g), sem0)
        pltpu.make_async_copy(obuf1, emb_slice(0), sem1).wait()
        compute_chunk(g + 1, obuf1)
        pltpu.async_copy(obuf1, emb_slice(g + 1), sem1)

    pltpu.make_async_copy(obuf0, emb_slice(0), sem0).wait()
    pltpu.make_async_copy(obuf1, emb_slice(0), sem1).wait()

    pltpu.sync_copy(widebuf, wide_hbm.at[pl.ds(base * 2, _BPW * 2)])


@functools.partial(
    pl.kernel,
    out_type=(
        jax.ShapeDtypeStruct((_B * _ROW,), jnp.float32),
        jax.ShapeDtypeStruct((_B * 2,), jnp.float32),
    ),
    mesh=plsc.VectorSubcoreMesh(core_axis_name="c", subcore_axis_name="s"),
    compiler_params=pltpu.CompilerParams(needs_layout_passes=False),
    scratch_types=[
        pltpu.VMEM((_BPW * _D,), jnp.float32),
        pltpu.VMEM((_DP, 128), jnp.float32),
        pltpu.VMEM((_DP * _EMB,), jnp.float32),
        pltpu.VMEM((_DP,), jnp.float32),
        pltpu.VMEM((_DP,), jnp.int32),
        pltpu.VMEM((_R * _ROW,), jnp.float32),
        pltpu.VMEM((_R * _ROW,), jnp.float32),
        pltpu.VMEM((_BPW * 2,), jnp.float32),
        pltpu.SMEM((_DP,), jnp.float32),
        pltpu.SMEM((_DP,), jnp.float32),
        pltpu.SMEM((_DP,), jnp.float32),
        pltpu.SemaphoreType.DMA,
        pltpu.SemaphoreType.DMA,
        pltpu.SemaphoreType.DMA,
    ],
)
def _wide_part_sc(x_hbm, w_hbm, v_hbm, vflat_hbm, idx_hbm, emb_hbm, wide_hbm,
                  xbuf, vbuf, vfbuf, wbuf, idxbuf, obuf0, obuf1, widebuf,
                  wsm, sbuf, qbuf, sem, sem0, sem1):
    _sc_body(x_hbm, w_hbm, v_hbm, vflat_hbm, idx_hbm, emb_hbm, wide_hbm,
             xbuf, vbuf, vfbuf, wbuf, idxbuf, obuf0, obuf1, widebuf,
             wsm, sbuf, qbuf, sem, sem0, sem1)


@jax.jit
def kernel(inputs, W, V, embedding_lookup_index):
    x_flat = jnp.reshape(inputs, (-1,))
    w_pad = jnp.pad(W, (0, _DP - _D))
    idx_pad = jnp.pad(embedding_lookup_index.astype(jnp.int32), (0, _DP - _D))
    v_pad = jnp.pad(jnp.reshape(V, (-1,)), (0, (_DP - _D) * _EMB))
    v_wide = jnp.pad(V, ((0, 0), (0, 128 - _EMB)))
    emb_flat, wide_flat = _wide_part_sc(x_flat, w_pad, v_wide, v_pad, idx_pad)
    wide_output = jnp.reshape(wide_flat, (_B, 2))
    embedded_fields = jnp.reshape(emb_flat, (_B, _D, _EMB))
    return (wide_output, embedded_fields)


# speedup vs baseline: 3.1023x; 3.1023x over previous
import functools

import jax
import jax.numpy as jnp
from jax import lax
from jax.experimental import pallas as pl
from jax.experimental.pallas import tpu as pltpu
from jax.experimental.pallas import tpu_sc as plsc

_B = 16384
_D = 26
_EMB = 64
_NC = 2
_NS = 16
_NW = _NC * _NS
_BPW = _B // _NW
_DP = 32
_NT = _D * (_EMB // 8)


def _sc_body(x_hbm, w_hbm, v_hbm, vflat_hbm, idx_hbm, emb_hbm, wide_hbm,
             xbuf, xt, vbuf, vfbuf, wbuf, idxbuf, obuf0, obuf1, widebuf,
             wsm, sbuf, qbuf, sem, sem0, sem1):
    cid = lax.axis_index("c")
    sid = lax.axis_index("s")
    wid = sid * _NC + cid
    base = wid * _BPW

    pltpu.sync_copy(x_hbm.at[pl.ds(base * _D, _BPW * _D)], xbuf)
    pltpu.sync_copy(w_hbm, wbuf)
    pltpu.sync_copy(idx_hbm, idxbuf)
    pltpu.sync_copy(vflat_hbm, vfbuf)
    pltpu.async_copy(v_hbm.at[idxbuf], vbuf, sem).wait()

    lanes = lax.iota(jnp.int32, 16)
    wv0 = wbuf[pl.ds(0, 16)]
    wv1 = wbuf[pl.ds(16, 16)]
    s_lo = jnp.zeros((16,), jnp.float32)
    s_hi = jnp.zeros((16,), jnp.float32)
    q_lo = jnp.zeros((16,), jnp.float32)
    q_hi = jnp.zeros((16,), jnp.float32)
    for e in range(_EMB):
        ve_lo = plsc.load_gather(vfbuf, [lanes * _EMB + e])
        ve_hi = plsc.load_gather(vfbuf, [(lanes + 16) * _EMB + e])
        s_lo = s_lo + ve_lo
        s_hi = s_hi + ve_hi
        q_lo = q_lo + ve_lo * ve_lo
        q_hi = q_hi + ve_hi * ve_hi
    for d in range(_D):
        sbuf[d] = s_lo[d] if d < 16 else s_hi[d - 16]
        qbuf[d] = q_lo[d] if d < 16 else q_hi[d - 16]
        wsm[d] = wv0[d] if d < 16 else wv1[d - 16]
    zero16 = jnp.zeros((16,), jnp.float32)

    @pl.loop(0, _BPW // 16)
    def _tr(bg):
        cols = (bg * 16 + lanes) * _D
        for d in range(_D):
            xt[d, pl.ds(bg * 16, 16)] = plsc.load_gather(xbuf, [cols + d])

    def block(d, et, vrow, half, obuf, bsem):
        vs = [vrow.at[jnp.full((16,), half * 8 + ei, jnp.int32)]
              .get(mode="promise_in_bounds") for ei in range(8)]
        for bg in range(_BPW // 16):
            xv = xt[d, pl.ds(bg * 16, 16)]
            for ei in range(8):
                obuf[ei, pl.ds(bg * 16, 16)] = xv * vs[ei]
        pltpu.async_copy(
            obuf, emb_hbm.at[d, pl.ds(et * 8, 8), pl.ds(base, _BPW)], bsem)

    def drain(obuf, bsem):
        pltpu.make_async_copy(
            obuf, emb_hbm.at[0, pl.ds(0, 8), pl.ds(0, _BPW)], bsem).wait()

    vrow00 = vbuf[0, pl.ds(0, 16)]
    block(0, 0, vrow00, 0, obuf0, sem0)
    block(0, 1, vrow00, 1, obuf1, sem1)

    @pl.loop(1, _NT // 2)
    def _pair(p):
        t = 2 * p
        d = t // 8
        et0 = t - 8 * d
        vrow = vbuf[d, pl.ds(et0 * 8, 16)]
        drain(obuf0, sem0)
        block(d, et0, vrow, 0, obuf0, sem0)
        drain(obuf1, sem1)
        block(d, et0 + 1, vrow, 1, obuf1, sem1)

    drain(obuf0, sem0)
    drain(obuf1, sem1)

    @pl.loop(0, _BPW // 16)
    def _grp(grp):
        rows = grp * 16 + lanes

        @pl.loop(0, _D, init_carry=(zero16, zero16, zero16))
        def _field(d, carry):
            o1, sv, qv = carry
            xv = xt[d, pl.ds(grp * 16, 16)]
            o1 = o1 + xv * wsm[d]
            sv = sv + xv * sbuf[d]
            qv = qv + (xv * xv) * qbuf[d]
            return (o1, sv, qv)

        o1, sv, qv = _field
        o2 = 0.5 * (sv * sv - qv)
        plsc.store_scatter(widebuf, [rows * 2], o1)
        plsc.store_scatter(widebuf, [rows * 2 + 1], o2)

    pltpu.sync_copy(widebuf, wide_hbm.at[pl.ds(base * 2, _BPW * 2)])


@functools.partial(
    pl.kernel,
    out_type=(
        jax.ShapeDtypeStruct((_D, _EMB, _B), jnp.float32),
        jax.ShapeDtypeStruct((_B * 2,), jnp.float32),
    ),
    mesh=plsc.VectorSubcoreMesh(core_axis_name="c", subcore_axis_name="s"),
    compiler_params=pltpu.CompilerParams(needs_layout_passes=False),
    scratch_types=[
        pltpu.VMEM((_BPW * _D,), jnp.float32),
        pltpu.VMEM((_D, _BPW), jnp.float32),
        pltpu.VMEM((_DP, 128), jnp.float32),
        pltpu.VMEM((_DP * _EMB,), jnp.float32),
        pltpu.VMEM((_DP,), jnp.float32),
        pltpu.VMEM((_DP,), jnp.int32),
        pltpu.VMEM((8, _BPW), jnp.float32),
        pltpu.VMEM((8, _BPW), jnp.float32),
        pltpu.VMEM((_BPW * 2,), jnp.float32),
        pltpu.SMEM((_D,), jnp.float32),
        pltpu.SMEM((_D,), jnp.float32),
        pltpu.SMEM((_D,), jnp.float32),
        pltpu.SemaphoreType.DMA,
        pltpu.SemaphoreType.DMA,
        pltpu.SemaphoreType.DMA,
    ],
)
def _wide_part_sc(x_hbm, w_hbm, v_hbm, vflat_hbm, idx_hbm, emb_hbm, wide_hbm,
                  xbuf, xt, vbuf, vfbuf, wbuf, idxbuf, obuf0, obuf1, widebuf,
                  wsm, sbuf, qbuf, sem, sem0, sem1):
    _sc_body(x_hbm, w_hbm, v_hbm, vflat_hbm, idx_hbm, emb_hbm, wide_hbm,
             xbuf, xt, vbuf, vfbuf, wbuf, idxbuf, obuf0, obuf1, widebuf,
             wsm, sbuf, qbuf, sem, sem0, sem1)


@jax.jit
def kernel(inputs, W, V, embedding_lookup_index):
    x_flat = jnp.reshape(inputs, (-1,))
    w_pad = jnp.pad(W, (0, _DP - _D))
    idx_pad = jnp.pad(embedding_lookup_index.astype(jnp.int32), (0, _DP - _D))
    v_pad = jnp.pad(jnp.reshape(V, (-1,)), (0, (_DP - _D) * _EMB))
    v_wide = jnp.pad(V, ((0, 0), (0, 128 - _EMB)))
    emb_t, wide_flat = _wide_part_sc(x_flat, w_pad, v_wide, v_pad, idx_pad)
    wide_output = jnp.reshape(wide_flat, (_B, 2))
    embedded_fields = jnp.transpose(emb_t, (2, 0, 1))
    return (wide_output, embedded_fields)


# speedup vs baseline: 3.4277x; 1.1049x over previous
import functools

import jax
import jax.numpy as jnp
from jax import lax
from jax.experimental import pallas as pl
from jax.experimental.pallas import tpu as pltpu
from jax.experimental.pallas import tpu_sc as plsc

_B = 16384
_D = 26
_EMB = 64
_NC = 2
_NS = 16
_NW = _NC * _NS
_BPW = _B // _NW
_DP = 32
_NT = _D * (_EMB // 8)


def _sc_body(x_hbm, w_hbm, v_hbm, vflat_hbm, idx_hbm, emb_hbm, wide_hbm,
             xbuf, xt, vbuf, vfbuf, wbuf, idxbuf, obuf0, obuf1, widebuf,
             wsm, sbuf, qbuf, sem, sem0, sem1):
    cid = lax.axis_index("c")
    sid = lax.axis_index("s")
    wid = sid * _NC + cid
    base = wid * _BPW

    pltpu.sync_copy(x_hbm.at[pl.ds(base * _D, _BPW * _D)], xbuf)
    pltpu.sync_copy(w_hbm, wbuf)
    pltpu.sync_copy(idx_hbm, idxbuf)
    pltpu.sync_copy(vflat_hbm, vfbuf)
    pltpu.async_copy(v_hbm.at[idxbuf], vbuf, sem).wait()

    lanes = lax.iota(jnp.int32, 16)
    wv0 = wbuf[pl.ds(0, 16)]
    wv1 = wbuf[pl.ds(16, 16)]
    s_lo = jnp.zeros((16,), jnp.float32)
    s_hi = jnp.zeros((16,), jnp.float32)
    q_lo = jnp.zeros((16,), jnp.float32)
    q_hi = jnp.zeros((16,), jnp.float32)
    for e in range(_EMB):
        ve_lo = plsc.load_gather(vfbuf, [lanes * _EMB + e])
        ve_hi = plsc.load_gather(vfbuf, [(lanes + 16) * _EMB + e])
        s_lo = s_lo + ve_lo
        s_hi = s_hi + ve_hi
        q_lo = q_lo + ve_lo * ve_lo
        q_hi = q_hi + ve_hi * ve_hi
    for d in range(_D):
        sbuf[d] = s_lo[d] if d < 16 else s_hi[d - 16]
        qbuf[d] = q_lo[d] if d < 16 else q_hi[d - 16]
        wsm[d] = wv0[d] if d < 16 else wv1[d - 16]
    zero16 = jnp.zeros((16,), jnp.float32)

    @pl.loop(0, _BPW // 16)
    def _tr(bg):
        cols = (bg * 16 + lanes) * _D
        for d in range(_D):
            xt[d, pl.ds(bg * 16, 16)] = plsc.load_gather(xbuf, [cols + d])

    def block(d, obuf, bsem):
        @pl.loop(0, _EMB // 16)
        def _epair(ep):
            vrow = vbuf[d, pl.ds(ep * 16, 16)]
            vs = [vrow.at[jnp.full((16,), ei, jnp.int32)]
                  .get(mode="promise_in_bounds") for ei in range(16)]
            for bg in range(_BPW // 16):
                xv = xt[d, pl.ds(bg * 16, 16)]
                for ei in range(16):
                    obuf[ep * 16 + ei, pl.ds(bg * 16, 16)] = xv * vs[ei]
        pltpu.async_copy(
            obuf, emb_hbm.at[d, pl.ds(0, _EMB), pl.ds(base, _BPW)], bsem)

    def drain(obuf, bsem):
        pltpu.make_async_copy(
            obuf, emb_hbm.at[0, pl.ds(0, _EMB), pl.ds(0, _BPW)], bsem).wait()

    block(0, obuf0, sem0)
    block(1, obuf1, sem1)

    @pl.loop(1, _D // 2)
    def _pair(p):
        drain(obuf0, sem0)
        block(2 * p, obuf0, sem0)
        drain(obuf1, sem1)
        block(2 * p + 1, obuf1, sem1)

    drain(obuf0, sem0)
    drain(obuf1, sem1)

    @pl.loop(0, _BPW // 16)
    def _grp(grp):
        rows = grp * 16 + lanes

        @pl.loop(0, _D, init_carry=(zero16, zero16, zero16))
        def _field(d, carry):
            o1, sv, qv = carry
            xv = xt[d, pl.ds(grp * 16, 16)]
            o1 = o1 + xv * wsm[d]
            sv = sv + xv * sbuf[d]
            qv = qv + (xv * xv) * qbuf[d]
            return (o1, sv, qv)

        o1, sv, qv = _field
        o2 = 0.5 * (sv * sv - qv)
        plsc.store_scatter(widebuf, [rows * 2], o1)
        plsc.store_scatter(widebuf, [rows * 2 + 1], o2)

    pltpu.sync_copy(widebuf, wide_hbm.at[pl.ds(base * 2, _BPW * 2)])


@functools.partial(
    pl.kernel,
    out_type=(
        jax.ShapeDtypeStruct((_D, _EMB, _B), jnp.float32),
        jax.ShapeDtypeStruct((_B * 2,), jnp.float32),
    ),
    mesh=plsc.VectorSubcoreMesh(core_axis_name="c", subcore_axis_name="s"),
    compiler_params=pltpu.CompilerParams(needs_layout_passes=False),
    scratch_types=[
        pltpu.VMEM((_BPW * _D,), jnp.float32),
        pltpu.VMEM((_D, _BPW), jnp.float32),
        pltpu.VMEM((_DP, 128), jnp.float32),
        pltpu.VMEM((_DP * _EMB,), jnp.float32),
        pltpu.VMEM((_DP,), jnp.float32),
        pltpu.VMEM((_DP,), jnp.int32),
        pltpu.VMEM((_EMB, _BPW), jnp.float32),
        pltpu.VMEM((_EMB, _BPW), jnp.float32),
        pltpu.VMEM((_BPW * 2,), jnp.float32),
        pltpu.SMEM((_D,), jnp.float32),
        pltpu.SMEM((_D,), jnp.float32),
        pltpu.SMEM((_D,), jnp.float32),
        pltpu.SemaphoreType.DMA,
        pltpu.SemaphoreType.DMA,
        pltpu.SemaphoreType.DMA,
    ],
)
def _wide_part_sc(x_hbm, w_hbm, v_hbm, vflat_hbm, idx_hbm, emb_hbm, wide_hbm,
                  xbuf, xt, vbuf, vfbuf, wbuf, idxbuf, obuf0, obuf1, widebuf,
                  wsm, sbuf, qbuf, sem, sem0, sem1):
    _sc_body(x_hbm, w_hbm, v_hbm, vflat_hbm, idx_hbm, emb_hbm, wide_hbm,
             xbuf, xt, vbuf, vfbuf, wbuf, idxbuf, obuf0, obuf1, widebuf,
             wsm, sbuf, qbuf, sem, sem0, sem1)


@jax.jit
def kernel(inputs, W, V, embedding_lookup_index):
    x_flat = jnp.reshape(inputs, (-1,))
    w_pad = jnp.pad(W, (0, _DP - _D))
    idx_pad = jnp.pad(embedding_lookup_index.astype(jnp.int32), (0, _DP - _D))
    v_pad = jnp.pad(jnp.reshape(V, (-1,)), (0, (_DP - _D) * _EMB))
    v_wide = jnp.pad(V, ((0, 0), (0, 128 - _EMB)))
    emb_t, wide_flat = _wide_part_sc(x_flat, w_pad, v_wide, v_pad, idx_pad)
    wide_output = jnp.reshape(wide_flat, (_B, 2))
    embedded_fields = jnp.transpose(emb_t, (2, 0, 1))
    return (wide_output, embedded_fields)


# speedup vs baseline: 3.5878x; 1.0467x over previous
import functools

import jax
import jax.numpy as jnp
from jax import lax
from jax.experimental import pallas as pl
from jax.experimental.pallas import tpu as pltpu
from jax.experimental.pallas import tpu_sc as plsc

_B = 16384
_D = 26
_EMB = 64
_NC = 2
_NS = 16
_NW = _NC * _NS
_BPW = _B // _NW
_DP = 32
_NT = _D * (_EMB // 8)


def _sc_body(xt_hbm, w_hbm, v_hbm, vflat_hbm, idx_hbm, emb_hbm, wide_hbm,
             xt, vbuf, vfbuf, wbuf, idxbuf, obuf0, obuf1, widebuf,
             wsm, sbuf, qbuf, sem, sem0, sem1):
    cid = lax.axis_index("c")
    sid = lax.axis_index("s")
    wid = sid * _NC + cid
    base = wid * _BPW

    for r0, nr in ((0, 8), (8, 8), (16, 8), (24, 2)):
        pltpu.sync_copy(xt_hbm.at[pl.ds(r0, nr), pl.ds(base, _BPW)],
                        xt.at[pl.ds(r0, nr)])
    pltpu.sync_copy(w_hbm, wbuf)
    pltpu.sync_copy(idx_hbm, idxbuf)
    pltpu.sync_copy(vflat_hbm, vfbuf)
    pltpu.async_copy(v_hbm.at[idxbuf], vbuf, sem).wait()

    lanes = lax.iota(jnp.int32, 16)
    wv0 = wbuf[pl.ds(0, 16)]
    wv1 = wbuf[pl.ds(16, 16)]
    s_lo = jnp.zeros((16,), jnp.float32)
    s_hi = jnp.zeros((16,), jnp.float32)
    q_lo = jnp.zeros((16,), jnp.float32)
    q_hi = jnp.zeros((16,), jnp.float32)
    for e in range(_EMB):
        ve_lo = plsc.load_gather(vfbuf, [lanes * _EMB + e])
        ve_hi = plsc.load_gather(vfbuf, [(lanes + 16) * _EMB + e])
        s_lo = s_lo + ve_lo
        s_hi = s_hi + ve_hi
        q_lo = q_lo + ve_lo * ve_lo
        q_hi = q_hi + ve_hi * ve_hi
    for d in range(_D):
        sbuf[d] = s_lo[d] if d < 16 else s_hi[d - 16]
        qbuf[d] = q_lo[d] if d < 16 else q_hi[d - 16]
        wsm[d] = wv0[d] if d < 16 else wv1[d - 16]
    zero16 = jnp.zeros((16,), jnp.float32)

    def block(d, obuf, bsem):
        @pl.loop(0, _EMB // 16)
        def _epair(ep):
            vrow = vbuf[d, pl.ds(ep * 16, 16)]
            vs = [vrow.at[jnp.full((16,), ei, jnp.int32)]
                  .get(mode="promise_in_bounds") for ei in range(16)]
            for bg in range(_BPW // 16):
                xv = xt[d, pl.ds(bg * 16, 16)]
                for ei in range(16):
                    obuf[ep * 16 + ei, pl.ds(bg * 16, 16)] = xv * vs[ei]
        pltpu.async_copy(
            obuf, emb_hbm.at[d, pl.ds(0, _EMB), pl.ds(base, _BPW)], bsem)

    def drain(obuf, bsem):
        pltpu.make_async_copy(
            obuf, emb_hbm.at[0, pl.ds(0, _EMB), pl.ds(0, _BPW)], bsem).wait()

    block(0, obuf0, sem0)
    block(1, obuf1, sem1)

    @pl.loop(1, _D // 2)
    def _pair(p):
        drain(obuf0, sem0)
        block(2 * p, obuf0, sem0)
        drain(obuf1, sem1)
        block(2 * p + 1, obuf1, sem1)

    drain(obuf0, sem0)
    drain(obuf1, sem1)

    @pl.loop(0, _BPW // 16)
    def _grp(grp):
        rows = grp * 16 + lanes

        @pl.loop(0, _D, init_carry=(zero16, zero16, zero16))
        def _field(d, carry):
            o1, sv, qv = carry
            xv = xt[d, pl.ds(grp * 16, 16)]
            o1 = o1 + xv * wsm[d]
            sv = sv + xv * sbuf[d]
            qv = qv + (xv * xv) * qbuf[d]
            return (o1, sv, qv)

        o1, sv, qv = _field
        o2 = 0.5 * (sv * sv - qv)
        plsc.store_scatter(widebuf, [rows * 2], o1)
        plsc.store_scatter(widebuf, [rows * 2 + 1], o2)

    pltpu.sync_copy(widebuf, wide_hbm.at[pl.ds(base * 2, _BPW * 2)])


@functools.partial(
    pl.kernel,
    out_type=(
        jax.ShapeDtypeStruct((_D, _EMB, _B), jnp.float32),
        jax.ShapeDtypeStruct((_B * 2,), jnp.float32),
    ),
    mesh=plsc.VectorSubcoreMesh(core_axis_name="c", subcore_axis_name="s"),
    compiler_params=pltpu.CompilerParams(needs_layout_passes=False),
    scratch_types=[
        pltpu.VMEM((_DP, _BPW), jnp.float32),
        pltpu.VMEM((_DP, 128), jnp.float32),
        pltpu.VMEM((_DP * _EMB,), jnp.float32),
        pltpu.VMEM((_DP,), jnp.float32),
        pltpu.VMEM((_DP,), jnp.int32),
        pltpu.VMEM((_EMB, _BPW), jnp.float32),
        pltpu.VMEM((_EMB, _BPW), jnp.float32),
        pltpu.VMEM((_BPW * 2,), jnp.float32),
        pltpu.SMEM((_D,), jnp.float32),
        pltpu.SMEM((_D,), jnp.float32),
        pltpu.SMEM((_D,), jnp.float32),
        pltpu.SemaphoreType.DMA,
        pltpu.SemaphoreType.DMA,
        pltpu.SemaphoreType.DMA,
    ],
)
def _wide_part_sc(xt_hbm, w_hbm, v_hbm, vflat_hbm, idx_hbm, emb_hbm, wide_hbm,
                  xt, vbuf, vfbuf, wbuf, idxbuf, obuf0, obuf1, widebuf,
                  wsm, sbuf, qbuf, sem, sem0, sem1):
    _sc_body(xt_hbm, w_hbm, v_hbm, vflat_hbm, idx_hbm, emb_hbm, wide_hbm,
             xt, vbuf, vfbuf, wbuf, idxbuf, obuf0, obuf1, widebuf,
             wsm, sbuf, qbuf, sem, sem0, sem1)


@jax.jit
def kernel(inputs, W, V, embedding_lookup_index):
    x_t = jnp.transpose(inputs)
    w_pad = jnp.pad(W, (0, _DP - _D))
    idx_pad = jnp.pad(embedding_lookup_index.astype(jnp.int32), (0, _DP - _D))
    v_pad = jnp.pad(jnp.reshape(V, (-1,)), (0, (_DP - _D) * _EMB))
    v_wide = jnp.pad(V, ((0, 0), (0, 128 - _EMB)))
    emb_t, wide_flat = _wide_part_sc(x_t, w_pad, v_wide, v_pad, idx_pad)
    wide_output = jnp.reshape(wide_flat, (_B, 2))
    embedded_fields = jnp.transpose(emb_t, (2, 0, 1))
    return (wide_output, embedded_fields)


# speedup vs baseline: 4.2399x; 1.1817x over previous
import functools

import jax
import jax.numpy as jnp
from jax import lax
from jax.experimental import pallas as pl
from jax.experimental.pallas import tpu as pltpu
from jax.experimental.pallas import tpu_sc as plsc

_B = 16384
_D = 26
_EMB = 64
_NC = 2
_NS = 16
_NW = _NC * _NS
_BPW = _B // _NW
_DP = 32
_NT = _D * (_EMB // 8)


def _sc_body(xt_hbm, w_hbm, v_hbm, vflat_hbm, idx_hbm, emb_hbm, wide_hbm,
             xt, vbuf, vfbuf, wbuf, idxbuf, obuf0, obuf1, widebuf,
             wsm, sbuf, qbuf, sem, sem0, sem1):
    cid = lax.axis_index("c")
    sid = lax.axis_index("s")
    wid = sid * _NC + cid
    base = wid * _BPW

    for r0, nr in ((0, 8), (8, 8), (16, 8), (24, 2)):
        pltpu.sync_copy(xt_hbm.at[pl.ds(r0, nr), pl.ds(base, _BPW)],
                        xt.at[pl.ds(r0, nr)])
    pltpu.sync_copy(w_hbm, wbuf)
    pltpu.sync_copy(idx_hbm, idxbuf)
    pltpu.sync_copy(vflat_hbm, vfbuf)
    pltpu.async_copy(v_hbm.at[idxbuf], vbuf, sem).wait()

    lanes = lax.iota(jnp.int32, 16)
    wv0 = wbuf[pl.ds(0, 16)]
    wv1 = wbuf[pl.ds(16, 16)]
    s_lo = jnp.zeros((16,), jnp.float32)
    s_hi = jnp.zeros((16,), jnp.float32)
    q_lo = jnp.zeros((16,), jnp.float32)
    q_hi = jnp.zeros((16,), jnp.float32)
    for e in range(_EMB):
        ve_lo = plsc.load_gather(vfbuf, [lanes * _EMB + e])
        ve_hi = plsc.load_gather(vfbuf, [(lanes + 16) * _EMB + e])
        s_lo = s_lo + ve_lo
        s_hi = s_hi + ve_hi
        q_lo = q_lo + ve_lo * ve_lo
        q_hi = q_hi + ve_hi * ve_hi
    for d in range(_D):
        sbuf[d] = s_lo[d] if d < 16 else s_hi[d - 16]
        qbuf[d] = q_lo[d] if d < 16 else q_hi[d - 16]
        wsm[d] = wv0[d] if d < 16 else wv1[d - 16]
    zero16 = jnp.zeros((16,), jnp.float32)

    def block(d, obuf, bsem):
        @pl.loop(0, _EMB // 16)
        def _epair(ep):
            vrow = vbuf[d, pl.ds(ep * 16, 16)]
            vs = [vrow.at[jnp.full((16,), ei, jnp.int32)]
                  .get(mode="promise_in_bounds") for ei in range(16)]
            for bg in range(_BPW // 16):
                xv = xt[d, pl.ds(bg * 16, 16)]
                for ei in range(16):
                    obuf[ep * 16 + ei, pl.ds(bg * 16, 16)] = xv * vs[ei]
        pltpu.async_copy(
            obuf, emb_hbm.at[d, pl.ds(0, _EMB), pl.ds(base, _BPW)], bsem)

    def drain(obuf, bsem):
        pltpu.make_async_copy(
            obuf, emb_hbm.at[0, pl.ds(0, _EMB), pl.ds(0, _BPW)], bsem).wait()

    block(0, obuf0, sem0)
    block(1, obuf1, sem1)

    @pl.loop(1, _D // 2)
    def _pair(p):
        drain(obuf0, sem0)
        block(2 * p, obuf0, sem0)
        drain(obuf1, sem1)
        block(2 * p + 1, obuf1, sem1)

    @pl.loop(0, _BPW // 16)
    def _grp(grp):
        pos = (grp // 8) * 256 + (grp % 8) * 16 + lanes

        @pl.loop(0, _D, init_carry=(zero16, zero16, zero16))
        def _field(d, carry):
            o1, sv, qv = carry
            xv = xt[d, pl.ds(grp * 16, 16)]
            o1 = o1 + xv * wsm[d]
            sv = sv + xv * sbuf[d]
            qv = qv + (xv * xv) * qbuf[d]
            return (o1, sv, qv)

        o1, sv, qv = _field
        o2 = 0.5 * (sv * sv - qv)
        plsc.store_scatter(widebuf, [pos], o1)
        plsc.store_scatter(widebuf, [pos + 128], o2)

    pltpu.sync_copy(widebuf, wide_hbm.at[pl.ds(base * 2, _BPW * 2)])
    drain(obuf0, sem0)
    drain(obuf1, sem1)


@functools.partial(
    pl.kernel,
    out_type=(
        jax.ShapeDtypeStruct((_D, _EMB, _B), jnp.float32),
        jax.ShapeDtypeStruct((_B * 2,), jnp.float32),
    ),
    mesh=plsc.VectorSubcoreMesh(core_axis_name="c", subcore_axis_name="s"),
    compiler_params=pltpu.CompilerParams(needs_layout_passes=False),
    scratch_types=[
        pltpu.VMEM((_DP, _BPW), jnp.float32),
        pltpu.VMEM((_DP, 128), jnp.float32),
        pltpu.VMEM((_DP * _EMB,), jnp.float32),
        pltpu.VMEM((_DP,), jnp.float32),
        pltpu.VMEM((_DP,), jnp.int32),
        pltpu.VMEM((_EMB, _BPW), jnp.float32),
        pltpu.VMEM((_EMB, _BPW), jnp.float32),
        pltpu.VMEM((_BPW * 2,), jnp.float32),
        pltpu.SMEM((_D,), jnp.float32),
        pltpu.SMEM((_D,), jnp.float32),
        pltpu.SMEM((_D,), jnp.float32),
        pltpu.SemaphoreType.DMA,
        pltpu.SemaphoreType.DMA,
        pltpu.SemaphoreType.DMA,
    ],
)
def _wide_part_sc(xt_hbm, w_hbm, v_hbm, vflat_hbm, idx_hbm, emb_hbm, wide_hbm,
                  xt, vbuf, vfbuf, wbuf, idxbuf, obuf0, obuf1, widebuf,
                  wsm, sbuf, qbuf, sem, sem0, sem1):
    _sc_body(xt_hbm, w_hbm, v_hbm, vflat_hbm, idx_hbm, emb_hbm, wide_hbm,
             xt, vbuf, vfbuf, wbuf, idxbuf, obuf0, obuf1, widebuf,
             wsm, sbuf, qbuf, sem, sem0, sem1)


@jax.jit
def kernel(inputs, W, V, embedding_lookup_index):
    x_t = jnp.transpose(inputs)
    w_pad = jnp.pad(W, (0, _DP - _D))
    idx_pad = jnp.pad(embedding_lookup_index.astype(jnp.int32), (0, _DP - _D))
    v_pad = jnp.pad(jnp.reshape(V, (-1,)), (0, (_DP - _D) * _EMB))
    v_wide = jnp.pad(V, ((0, 0), (0, 128 - _EMB)))
    emb_t, wide_flat = _wide_part_sc(x_t, w_pad, v_wide, v_pad, idx_pad)
    wide_output = jnp.reshape(
        jnp.transpose(jnp.reshape(wide_flat, (_B // 128, 2, 128)), (0, 2, 1)),
        (_B, 2))
    embedded_fields = jnp.transpose(emb_t, (2, 0, 1))
    return (wide_output, embedded_fields)


# speedup vs baseline: 5.1778x; 1.2212x over previous
import functools

import jax
import jax.numpy as jnp
from jax import lax
from jax.experimental import pallas as pl
from jax.experimental.pallas import tpu as pltpu
from jax.experimental.pallas import tpu_sc as plsc

_B = 16384
_D = 26
_EMB = 64
_NC = 2
_NS = 16
_NW = _NC * _NS
_BPW = _B // _NW
_DP = 32
_NT = _D * (_EMB // 8)


def _sc_body(xt_hbm, w_hbm, v_hbm, vflat_hbm, idx_hbm, emb_hbm, wide_hbm,
             xt, vbuf, vfbuf, wbuf, idxbuf, obuf0, obuf1, widebuf,
             wsm, sbuf, qbuf, sem, sem0, sem1):
    cid = lax.axis_index("c")
    sid = lax.axis_index("s")
    wid = sid * _NC + cid
    base = wid * _BPW

    for r0, nr in ((0, 8), (8, 8), (16, 8), (24, 2)):
        pltpu.sync_copy(xt_hbm.at[pl.ds(r0, nr), pl.ds(base, _BPW)],
                        xt.at[pl.ds(r0, nr)])
    pltpu.sync_copy(w_hbm, wbuf.at[pl.ds(0, _D)])
    pltpu.sync_copy(idx_hbm, idxbuf)
    pltpu.sync_copy(vflat_hbm, vfbuf)
    pltpu.async_copy(v_hbm.at[idxbuf], vbuf, sem).wait()

    lanes = lax.iota(jnp.int32, 16)
    wv0 = wbuf[pl.ds(0, 16)]
    wv1 = wbuf[pl.ds(16, 16)]
    s_lo = jnp.zeros((16,), jnp.float32)
    s_hi = jnp.zeros((16,), jnp.float32)
    q_lo = jnp.zeros((16,), jnp.float32)
    q_hi = jnp.zeros((16,), jnp.float32)
    for e in range(_EMB):
        ve_lo = plsc.load_gather(vfbuf, [lanes * _EMB + e])
        ve_hi = plsc.load_gather(
            vfbuf, [jnp.minimum((lanes + 16) * _EMB + e, _D * _EMB - 1)])
        s_lo = s_lo + ve_lo
        s_hi = s_hi + ve_hi
        q_lo = q_lo + ve_lo * ve_lo
        q_hi = q_hi + ve_hi * ve_hi
    for d in range(_D):
        sbuf[d] = s_lo[d] if d < 16 else s_hi[d - 16]
        qbuf[d] = q_lo[d] if d < 16 else q_hi[d - 16]
        wsm[d] = wv0[d] if d < 16 else wv1[d - 16]
    zero16 = jnp.zeros((16,), jnp.float32)

    def block(d, obuf, bsem):
        @pl.loop(0, _EMB // 16)
        def _epair(ep):
            vrow = vbuf[d, pl.ds(ep * 16, 16)]
            vs = [vrow.at[jnp.full((16,), ei, jnp.int32)]
                  .get(mode="promise_in_bounds") for ei in range(16)]
            for bg in range(_BPW // 16):
                xv = xt[d, pl.ds(bg * 16, 16)]
                for ei in range(16):
                    obuf[ep * 16 + ei, pl.ds(bg * 16, 16)] = xv * vs[ei]
        pltpu.async_copy(
            obuf, emb_hbm.at[d, pl.ds(0, _EMB), pl.ds(base, _BPW)], bsem)

    def drain(obuf, bsem):
        pltpu.make_async_copy(
            obuf, emb_hbm.at[0, pl.ds(0, _EMB), pl.ds(0, _BPW)], bsem).wait()

    block(0, obuf0, sem0)
    block(1, obuf1, sem1)

    @pl.loop(1, _D // 2)
    def _pair(p):
        drain(obuf0, sem0)
        block(2 * p, obuf0, sem0)
        drain(obuf1, sem1)
        block(2 * p + 1, obuf1, sem1)

    @pl.loop(0, _BPW // 16)
    def _grp(grp):
        pos = (grp // 8) * 256 + (grp % 8) * 16 + lanes

        @pl.loop(0, _D, init_carry=(zero16, zero16, zero16))
        def _field(d, carry):
            o1, sv, qv = carry
            xv = xt[d, pl.ds(grp * 16, 16)]
            o1 = o1 + xv * wsm[d]
            sv = sv + xv * sbuf[d]
            qv = qv + (xv * xv) * qbuf[d]
            return (o1, sv, qv)

        o1, sv, qv = _field
        o2 = 0.5 * (sv * sv - qv)
        plsc.store_scatter(widebuf, [pos], o1)
        plsc.store_scatter(widebuf, [pos + 128], o2)

    pltpu.sync_copy(widebuf, wide_hbm.at[pl.ds(base * 2, _BPW * 2)])
    drain(obuf0, sem0)
    drain(obuf1, sem1)


@functools.partial(
    pl.kernel,
    out_type=(
        jax.ShapeDtypeStruct((_D, _EMB, _B), jnp.float32),
        jax.ShapeDtypeStruct((_B * 2,), jnp.float32),
    ),
    mesh=plsc.VectorSubcoreMesh(core_axis_name="c", subcore_axis_name="s"),
    compiler_params=pltpu.CompilerParams(needs_layout_passes=False),
    scratch_types=[
        pltpu.VMEM((_DP, _BPW), jnp.float32),
        pltpu.VMEM((_D, 128), jnp.float32),
        pltpu.VMEM((_D * _EMB,), jnp.float32),
        pltpu.VMEM((_DP,), jnp.float32),
        pltpu.VMEM((_D,), jnp.int32),
        pltpu.VMEM((_EMB, _BPW), jnp.float32),
        pltpu.VMEM((_EMB, _BPW), jnp.float32),
        pltpu.VMEM((_BPW * 2,), jnp.float32),
        pltpu.SMEM((_D,), jnp.float32),
        pltpu.SMEM((_D,), jnp.float32),
        pltpu.SMEM((_D,), jnp.float32),
        pltpu.SemaphoreType.DMA,
        pltpu.SemaphoreType.DMA,
        pltpu.SemaphoreType.DMA,
    ],
)
def _wide_part_sc(xt_hbm, w_hbm, v_hbm, vflat_hbm, idx_hbm, emb_hbm, wide_hbm,
                  xt, vbuf, vfbuf, wbuf, idxbuf, obuf0, obuf1, widebuf,
                  wsm, sbuf, qbuf, sem, sem0, sem1):
    _sc_body(xt_hbm, w_hbm, v_hbm, vflat_hbm, idx_hbm, emb_hbm, wide_hbm,
             xt, vbuf, vfbuf, wbuf, idxbuf, obuf0, obuf1, widebuf,
             wsm, sbuf, qbuf, sem, sem0, sem1)


@jax.jit
def kernel(inputs, W, V, embedding_lookup_index):
    x_t = jnp.transpose(inputs)
    idx32 = embedding_lookup_index.astype(jnp.int32)
    v_flat = jnp.reshape(V, (-1,))
    v_wide = jnp.pad(V, ((0, 0), (0, 128 - _EMB)))
    emb_t, wide_flat = _wide_part_sc(x_t, W, v_wide, v_flat, idx32)
    wide_output = jnp.reshape(
        jnp.transpose(jnp.reshape(wide_flat, (_B // 128, 2, 128)), (0, 2, 1)),
        (_B, 2))
    embedded_fields = jnp.transpose(emb_t, (2, 0, 1))
    return (wide_output, embedded_fields)
